# Initial kernel scaffold; baseline (speedup 1.0000x reference)
import functools

import jax
import jax.numpy as jnp
from jax import lax
from jax.experimental import pallas as pl
from jax.experimental.pallas import tpu as pltpu
from jax.experimental.pallas import tpu_sc as plsc

N = 10000
D = 128
E = 320000
NC = 2
NS = 16
NW = NC * NS
CHUNK = 128
CPW = 79
EP = NW * CPW * CHUNK
NPAD = N + 16
ROWS_INIT = NPAD // NS
ROWS_OUT = N // NS
RB = 1000

_vmesh = plsc.VectorSubcoreMesh(core_axis_name="c", subcore_axis_name="s")


@functools.partial(
    pl.kernel,
    out_type=[jax.ShapeDtypeStruct((N, 8), jnp.float32),
              jax.ShapeDtypeStruct((N, 8), jnp.float32)],
    mesh=_vmesh,
    scratch_types=[
        pltpu.VMEM_SHARED((NPAD, 8), jnp.float32),
        pltpu.VMEM((CPW, CHUNK), jnp.int32),
        pltpu.VMEM((CHUNK, 8), jnp.float32),
    ],
)
def _deg_kernel(dst_hbm, ones_hbm, zeros_hbm, d0_hbm, d1_hbm,
                deg_sh, idx_v, ones_v):
    c = lax.axis_index("c")
    s = lax.axis_index("s")
    w = c * NS + s
    pltpu.sync_copy(zeros_hbm.at[pl.ds(s * ROWS_INIT, ROWS_INIT)],
                    deg_sh.at[pl.ds(s * ROWS_INIT, ROWS_INIT)])
    pltpu.sync_copy(ones_hbm, ones_v)
    pltpu.sync_copy(dst_hbm.at[w], idx_v)
    plsc.subcore_barrier()

    @pl.loop(0, CPW)
    def _(j):
        pltpu.sync_copy(ones_v, deg_sh.at[idx_v.at[j]], add=True)

    plsc.subcore_barrier()

    @pl.when(c == 0)
    def _():
        pltpu.sync_copy(deg_sh.at[pl.ds(s * ROWS_OUT, ROWS_OUT)],
                        d0_hbm.at[pl.ds(s * ROWS_OUT, ROWS_OUT)])

    @pl.when(c == 1)
    def _():
        pltpu.sync_copy(deg_sh.at[pl.ds(s * ROWS_OUT, ROWS_OUT)],
                        d1_hbm.at[pl.ds(s * ROWS_OUT, ROWS_OUT)])


@functools.partial(
    pl.kernel,
    out_type=[jax.ShapeDtypeStruct((N, D), jnp.float32),
              jax.ShapeDtypeStruct((N, D), jnp.float32)],
    mesh=_vmesh,
    scratch_types=[
        pltpu.VMEM_SHARED((NPAD, D), jnp.float32),
        pltpu.VMEM((CPW, CHUNK), jnp.int32),
        pltpu.VMEM((CPW, CHUNK), jnp.int32),
        pltpu.VMEM((CHUNK, D), jnp.float32),
    ],
)
def _agg_kernel(g_hbm, src_hbm, dst_hbm, zeros_hbm, a0_hbm, a1_hbm,
                agg_sh, src_v, dst_v, rows_v):
    c = lax.axis_index("c")
    s = lax.axis_index("s")
    w = c * NS + s
    pltpu.sync_copy(zeros_hbm.at[pl.ds(s * ROWS_INIT, ROWS_INIT)],
                    agg_sh.at[pl.ds(s * ROWS_INIT, ROWS_INIT)])
    pltpu.sync_copy(src_hbm.at[w], src_v)
    pltpu.sync_copy(dst_hbm.at[w], dst_v)
    plsc.subcore_barrier()

    @pl.loop(0, CPW)
    def _(j):
        pltpu.sync_copy(g_hbm.at[src_v.at[j]], rows_v)
        pltpu.sync_copy(rows_v, agg_sh.at[dst_v.at[j]], add=True)

    plsc.subcore_barrier()

    @pl.when(c == 0)
    def _():
        pltpu.sync_copy(agg_sh.at[pl.ds(s * ROWS_OUT, ROWS_OUT)],
                        a0_hbm.at[pl.ds(s * ROWS_OUT, ROWS_OUT)])

    @pl.when(c == 1)
    def _():
        pltpu.sync_copy(agg_sh.at[pl.ds(s * ROWS_OUT, ROWS_OUT)],
                        a1_hbm.at[pl.ds(s * ROWS_OUT, ROWS_OUT)])


def _mm_scale(x, W, d0, d1):
    def body(x_ref, w_ref, d0_ref, d1_ref, g_ref, dis_ref):
        deg = d0_ref[...] + d1_ref[...] + 1.0
        dis = lax.rsqrt(deg)
        h = jnp.dot(x_ref[...], w_ref[...], preferred_element_type=jnp.float32)
        g_ref[...] = h * dis
        dis_ref[...] = dis

    return pl.pallas_call(
        body,
        grid=(N // RB,),
        in_specs=[pl.BlockSpec((RB, D), lambda i: (i, 0)),
                  pl.BlockSpec((D, D), lambda i: (0, 0)),
                  pl.BlockSpec((RB, 1), lambda i: (i, 0)),
                  pl.BlockSpec((RB, 1), lambda i: (i, 0))],
        out_specs=[pl.BlockSpec((RB, D), lambda i: (i, 0)),
                   pl.BlockSpec((RB, 1), lambda i: (i, 0))],
        out_shape=[jax.ShapeDtypeStruct((N, D), jnp.float32),
                   jax.ShapeDtypeStruct((N, 1), jnp.float32)],
    )(x, W, d0, d1)


def _layer2(a0, a1, g1, dis, b, W):
    def body(a0_ref, a1_ref, g1_ref, dis_ref, b_ref, w_ref, g2_ref):
        t = (a0_ref[...] + a1_ref[...] + g1_ref[...]) * dis_ref[...] + b_ref[...]
        h = jnp.maximum(t, 0.0)
        g2_ref[...] = jnp.dot(
            h, w_ref[...], preferred_element_type=jnp.float32) * dis_ref[...]

    return pl.pallas_call(
        body,
        grid=(N // RB,),
        in_specs=[pl.BlockSpec((RB, D), lambda i: (i, 0)),
                  pl.BlockSpec((RB, D), lambda i: (i, 0)),
                  pl.BlockSpec((RB, D), lambda i: (i, 0)),
                  pl.BlockSpec((RB, 1), lambda i: (i, 0)),
                  pl.BlockSpec((1, D), lambda i: (0, 0)),
                  pl.BlockSpec((D, D), lambda i: (0, 0))],
        out_specs=pl.BlockSpec((RB, D), lambda i: (i, 0)),
        out_shape=jax.ShapeDtypeStruct((N, D), jnp.float32),
    )(a0, a1, g1, dis, b, W)


def _final(a0, a1, g2, dis, b):
    def body(a0_ref, a1_ref, g2_ref, dis_ref, b_ref, o_ref):
        o_ref[...] = (a0_ref[...] + a1_ref[...] + g2_ref[...]) * dis_ref[...] \
            + b_ref[...]

    return pl.pallas_call(
        body,
        grid=(N // RB,),
        in_specs=[pl.BlockSpec((RB, D), lambda i: (i, 0)),
                  pl.BlockSpec((RB, D), lambda i: (i, 0)),
                  pl.BlockSpec((RB, D), lambda i: (i, 0)),
                  pl.BlockSpec((RB, 1), lambda i: (i, 0)),
                  pl.BlockSpec((1, D), lambda i: (0, 0))],
        out_specs=pl.BlockSpec((RB, D), lambda i: (i, 0)),
        out_shape=jax.ShapeDtypeStruct((N, D), jnp.float32),
    )(a0, a1, g2, dis, b)


def kernel(x, edge_index, W1, b1, W2, b2):
    src = edge_index[0].astype(jnp.int32)
    dst = edge_index[1].astype(jnp.int32)
    pad = EP - E
    src_p = jnp.concatenate(
        [src, jnp.zeros((pad,), jnp.int32)]).reshape(NW, CPW, CHUNK)
    dst_p = jnp.concatenate(
        [dst, jnp.full((pad,), N, jnp.int32)]).reshape(NW, CPW, CHUNK)
    ones8 = jnp.ones((CHUNK, 8), jnp.float32)
    zeros8 = jnp.zeros((NPAD, 8), jnp.float32)
    zerosD = jnp.zeros((NPAD, D), jnp.float32)

    d0, d1 = _deg_kernel(dst_p, ones8, zeros8)
    g1, dis = _mm_scale(x, W1, d0[:, 0:1], d1[:, 0:1])
    a0, a1 = _agg_kernel(g1, src_p, dst_p, zerosD)
    g2 = _layer2(a0, a1, g1, dis, b1.reshape(1, D), W2)
    c0, c1 = _agg_kernel(g2, src_p, dst_p, zerosD)
    return _final(c0, c1, g2, dis, b2.reshape(1, D))


# speedup vs baseline: 10.4893x; 10.4893x over previous
import functools

import jax
import jax.numpy as jnp
from jax import lax
from jax.experimental import pallas as pl
from jax.experimental.pallas import tpu as pltpu
from jax.experimental.pallas import tpu_sc as plsc

N = 10000
D = 128
E = 320000
NC = 2
NS = 16
NW = NC * NS
CHUNK = 128
CPW = 79
EPW = CPW * CHUNK
EP = NW * EPW
NPAD = 10240
ROWS = NPAD // NS
RB = 1000

_vmesh = plsc.VectorSubcoreMesh(core_axis_name="c", subcore_axis_name="s")


@functools.partial(
    pl.kernel,
    out_type=[jax.ShapeDtypeStruct((NPAD, D), jnp.float32),
              jax.ShapeDtypeStruct((NPAD, D), jnp.float32)],
    mesh=_vmesh,
    scratch_types=[
        pltpu.VMEM_SHARED((NPAD, D), jnp.float32),
        pltpu.VMEM((CHUNK,), jnp.int32),
        pltpu.VMEM((CHUNK, D), jnp.float32),
    ],
)
def _deg_kernel(dst_hbm, ones_hbm, zeros_hbm, d0_hbm, d1_hbm,
                deg_sh, idx_v, ones_v):
    c = lax.axis_index("c")
    s = lax.axis_index("s")
    base = (c * NS + s) * EPW
    pltpu.sync_copy(zeros_hbm.at[pl.ds(s * ROWS, ROWS)],
                    deg_sh.at[pl.ds(s * ROWS, ROWS)])
    pltpu.sync_copy(ones_hbm, ones_v)
    plsc.subcore_barrier()

    @pl.loop(0, CPW)
    def _(j):
        pltpu.sync_copy(dst_hbm.at[pl.ds(base + j * CHUNK, CHUNK)], idx_v)
        pltpu.sync_copy(ones_v, deg_sh.at[idx_v], add=True)

    plsc.subcore_barrier()

    @pl.when(c == 0)
    def _():
        pltpu.sync_copy(deg_sh.at[pl.ds(s * ROWS, ROWS)],
                        d0_hbm.at[pl.ds(s * ROWS, ROWS)])

    @pl.when(c == 1)
    def _():
        pltpu.sync_copy(deg_sh.at[pl.ds(s * ROWS, ROWS)],
                        d1_hbm.at[pl.ds(s * ROWS, ROWS)])


@functools.partial(
    pl.kernel,
    out_type=[jax.ShapeDtypeStruct((NPAD, D), jnp.float32),
              jax.ShapeDtypeStruct((NPAD, D), jnp.float32)],
    mesh=_vmesh,
    scratch_types=[
        pltpu.VMEM_SHARED((NPAD, D), jnp.float32),
        pltpu.VMEM((CHUNK,), jnp.int32),
        pltpu.VMEM((CHUNK,), jnp.int32),
        pltpu.VMEM((CHUNK, D), jnp.float32),
    ],
)
def _agg_kernel(g_hbm, src_hbm, dst_hbm, zeros_hbm, a0_hbm, a1_hbm,
                agg_sh, src_v, dst_v, rows_v):
    c = lax.axis_index("c")
    s = lax.axis_index("s")
    base = (c * NS + s) * EPW
    pltpu.sync_copy(zeros_hbm.at[pl.ds(s * ROWS, ROWS)],
                    agg_sh.at[pl.ds(s * ROWS, ROWS)])
    plsc.subcore_barrier()

    @pl.loop(0, CPW)
    def _(j):
        pltpu.sync_copy(src_hbm.at[pl.ds(base + j * CHUNK, CHUNK)], src_v)
        pltpu.sync_copy(dst_hbm.at[pl.ds(base + j * CHUNK, CHUNK)], dst_v)
        pltpu.sync_copy(g_hbm.at[src_v], rows_v)
        pltpu.sync_copy(rows_v, agg_sh.at[dst_v], add=True)

    plsc.subcore_barrier()

    @pl.when(c == 0)
    def _():
        pltpu.sync_copy(agg_sh.at[pl.ds(s * ROWS, ROWS)],
                        a0_hbm.at[pl.ds(s * ROWS, ROWS)])

    @pl.when(c == 1)
    def _():
        pltpu.sync_copy(agg_sh.at[pl.ds(s * ROWS, ROWS)],
                        a1_hbm.at[pl.ds(s * ROWS, ROWS)])


def _mm_scale(x, W, d0w, d1w):
    def body(x_ref, w_ref, d0_ref, d1_ref, g_ref, dis_ref):
        dis = lax.rsqrt(d0_ref[...] + d1_ref[...] + 1.0)
        h = jnp.dot(x_ref[...], w_ref[...], preferred_element_type=jnp.float32)
        g_ref[...] = h * dis
        dis_ref[...] = dis

    return pl.pallas_call(
        body,
        grid=(N // RB,),
        in_specs=[pl.BlockSpec((RB, D), lambda i: (i, 0)),
                  pl.BlockSpec((D, D), lambda i: (0, 0)),
                  pl.BlockSpec((RB, D), lambda i: (i, 0)),
                  pl.BlockSpec((RB, D), lambda i: (i, 0))],
        out_specs=[pl.BlockSpec((RB, D), lambda i: (i, 0)),
                   pl.BlockSpec((RB, D), lambda i: (i, 0))],
        out_shape=[jax.ShapeDtypeStruct((N, D), jnp.float32),
                   jax.ShapeDtypeStruct((N, D), jnp.float32)],
    )(x, W, d0w, d1w)


def _layer2(a0, a1, g1, disw, b, W):
    def body(a0_ref, a1_ref, g1_ref, dis_ref, b_ref, w_ref, g2_ref):
        dis = dis_ref[...]
        t = (a0_ref[...] + a1_ref[...] + g1_ref[...]) * dis + b_ref[...]
        h = jnp.maximum(t, 0.0)
        g2_ref[...] = jnp.dot(
            h, w_ref[...], preferred_element_type=jnp.float32) * dis

    return pl.pallas_call(
        body,
        grid=(N // RB,),
        in_specs=[pl.BlockSpec((RB, D), lambda i: (i, 0)),
                  pl.BlockSpec((RB, D), lambda i: (i, 0)),
                  pl.BlockSpec((RB, D), lambda i: (i, 0)),
                  pl.BlockSpec((RB, D), lambda i: (i, 0)),
                  pl.BlockSpec((1, D), lambda i: (0, 0)),
                  pl.BlockSpec((D, D), lambda i: (0, 0))],
        out_specs=pl.BlockSpec((RB, D), lambda i: (i, 0)),
        out_shape=jax.ShapeDtypeStruct((N, D), jnp.float32),
    )(a0, a1, g1, disw, b, W)


def _final(a0, a1, g2, disw, b):
    def body(a0_ref, a1_ref, g2_ref, dis_ref, b_ref, o_ref):
        o_ref[...] = (a0_ref[...] + a1_ref[...] + g2_ref[...]) \
            * dis_ref[...] + b_ref[...]

    return pl.pallas_call(
        body,
        grid=(N // RB,),
        in_specs=[pl.BlockSpec((RB, D), lambda i: (i, 0)),
                  pl.BlockSpec((RB, D), lambda i: (i, 0)),
                  pl.BlockSpec((RB, D), lambda i: (i, 0)),
                  pl.BlockSpec((RB, D), lambda i: (i, 0)),
                  pl.BlockSpec((1, D), lambda i: (0, 0))],
        out_specs=pl.BlockSpec((RB, D), lambda i: (i, 0)),
        out_shape=jax.ShapeDtypeStruct((N, D), jnp.float32),
    )(a0, a1, g2, disw, b)


def kernel(x, edge_index, W1, b1, W2, b2):
    src = edge_index[0].astype(jnp.int32)
    dst = edge_index[1].astype(jnp.int32)
    pad = EP - E
    src_p = jnp.concatenate([src, jnp.zeros((pad,), jnp.int32)])
    dst_p = jnp.concatenate([dst, jnp.full((pad,), N, jnp.int32)])
    onesD = jnp.ones((CHUNK, D), jnp.float32)
    zerosD = jnp.zeros((NPAD, D), jnp.float32)

    d0, d1 = _deg_kernel(dst_p, onesD, zerosD)
    g1, disw = _mm_scale(x, W1, d0, d1)
    a0, a1 = _agg_kernel(g1, src_p, dst_p, zerosD)
    g2 = _layer2(a0, a1, g1, disw, b1.reshape(1, D), W2)
    c0, c1 = _agg_kernel(g2, src_p, dst_p, zerosD)
    return _final(c0, c1, g2, disw, b2.reshape(1, D))
